# baseline (speedup 1.0000x reference)
import functools

import jax
import jax.numpy as jnp
import numpy as np
from jax import lax
from jax.experimental import pallas as pl
from jax.experimental.pallas import tpu as pltpu
from jax.experimental.pallas import tpu_sc as plsc

N = 10000
E = 160000
IN = 128
HID = 128
HEADS = 8
FH = HID // HEADS
K = 8
OUT = 2

NP = 10240
EP = 165888
BLK = 200
GRID = N // BLK
NCORE = 2
NSUB = 16
CHUNK = 48
DP = 32
EPW = EP // (NCORE * NSUB)
NCHUNK = EPW // CHUNK
ROWS_PW = NP // NSUB

_IU, _JU = np.triu_indices(K, 1)
NPAIR = len(_IU)


def _ln(h, g, b):
    m = jnp.mean(h, axis=-1, keepdims=True)
    v = jnp.mean((h - m) * (h - m), axis=-1, keepdims=True)
    return (h - m) / jnp.sqrt(v + 1e-5) * g + b


def _elu(h):
    return jnp.where(h > 0, h, jnp.exp(h) - 1.0)


def _attn_tail(hp, acat, step, macc_ref, m_ref):
    alcat = jnp.dot(hp, acat, preferred_element_type=jnp.float32)
    bmax = jnp.max(alcat, axis=0, keepdims=True)

    @pl.when(step == 0)
    def _():
        macc_ref[...] = bmax

    @pl.when(step > 0)
    def _():
        macc_ref[...] = jnp.maximum(macc_ref[...], bmax)

    mc = macc_ref[...]
    ms = jnp.max(mc[:, 0:8]) + jnp.max(mc[:, 8:16])
    ms = jnp.maximum(ms, 0.0)
    m_ref[...] = jnp.full((1, 128), ms, jnp.float32)
    return alcat


def _tc_pre_body(x_ref, w1_ref, w2_ref, w3_ref, pi_ref, pj_ref, bc_ref,
                 gc_ref, bcn_ref, wg_ref, acat_ref,
                 h_ref, hp_ref, alcat_ref, m_ref, macc_ref):
    i = pl.program_id(0)
    x = x_ref[...]
    cosx = jnp.cos(jnp.pi * x)
    sinx = jnp.sin(jnp.pi * x)
    pre = jnp.dot(cosx, w1_ref[...], preferred_element_type=jnp.float32)
    pre += jnp.dot(sinx, w2_ref[...], preferred_element_type=jnp.float32)
    si = jnp.dot(sinx, pi_ref[...], preferred_element_type=jnp.float32)
    sj = jnp.dot(sinx, pj_ref[...], preferred_element_type=jnp.float32)
    pre += jnp.dot(si * sj, w3_ref[...], preferred_element_type=jnp.float32)
    pre += bc_ref[...]
    h = _elu(_ln(pre, gc_ref[...], bcn_ref[...]))
    h_ref[...] = h
    hp = jnp.dot(h, wg_ref[...], preferred_element_type=jnp.float32)
    hp_ref[...] = hp
    alcat_ref[...] = _attn_tail(hp, acat_ref[...], i, macc_ref, m_ref)


def _tc_mid_body(hprev_ref, num_ref, den_ref, bg_ref, gl_ref, bl_ref,
                 wg_ref, acat_ref,
                 h_ref, hp_ref, alcat_ref, m_ref, macc_ref):
    i = pl.program_id(0)
    nsum = num_ref[0] + num_ref[1]
    dsum = den_ref[0] + den_ref[1]
    gat = nsum / (dsum + 1e-16) + bg_ref[...]
    h = _elu(_ln(gat, gl_ref[...], bl_ref[...])) + hprev_ref[...]
    h_ref[...] = h
    hp = jnp.dot(h, wg_ref[...], preferred_element_type=jnp.float32)
    hp_ref[...] = hp
    alcat_ref[...] = _attn_tail(hp, acat_ref[...], i, macc_ref, m_ref)


def _tc_post_body(hprev_ref, num_ref, den_ref, bg_ref, gl_ref, bl_ref,
                  wm1_ref, bm1_ref, wm2_ref, bm2_ref, out_ref):
    nsum = num_ref[0] + num_ref[1]
    dsum = den_ref[0] + den_ref[1]
    gat = nsum / (dsum + 1e-16) + bg_ref[...]
    h = _elu(_ln(gat, gl_ref[...], bl_ref[...])) + hprev_ref[...]
    t = _elu(jnp.dot(h, wm1_ref[...], preferred_element_type=jnp.float32)
             + bm1_ref[...])
    out_ref[...] = (jnp.dot(t, wm2_ref[...], preferred_element_type=jnp.float32)
                    + bm2_ref[...])


def _row_spec():
    return pl.BlockSpec((BLK, 128), lambda i: (i, 0))


def _w_spec(r, c=128):
    return pl.BlockSpec((r, c), lambda i: (0, 0))


def _pair_spec():
    return pl.BlockSpec((2, BLK, 128), lambda i: (0, i, 0))


def _tc_pre(x, w1, w2, w3, pim, pjm, bc, gc, bcn, wg, acat):
    return pl.pallas_call(
        _tc_pre_body,
        grid=(GRID,),
        in_specs=[_row_spec()] + [_w_spec(128)] * 5 + [_w_spec(1)] * 3
                 + [_w_spec(128)] * 2,
        out_specs=[_row_spec(), _row_spec(), _row_spec(),
                   pl.BlockSpec((1, 128), lambda i: (0, 0))],
        out_shape=[jax.ShapeDtypeStruct((N, 128), jnp.float32),
                   jax.ShapeDtypeStruct((N, 128), jnp.float32),
                   jax.ShapeDtypeStruct((N, 128), jnp.float32),
                   jax.ShapeDtypeStruct((1, 128), jnp.float32)],
        scratch_shapes=[pltpu.VMEM((1, 128), jnp.float32)],
    )(x, w1, w2, w3, pim, pjm, bc, gc, bcn, wg, acat)


def _tc_mid(hprev, num2, den2, bg, gl, bl, wg, acat):
    return pl.pallas_call(
        _tc_mid_body,
        grid=(GRID,),
        in_specs=[_row_spec(), _pair_spec(), _pair_spec()]
                 + [_w_spec(1)] * 3 + [_w_spec(128)] * 2,
        out_specs=[_row_spec(), _row_spec(), _row_spec(),
                   pl.BlockSpec((1, 128), lambda i: (0, 0))],
        out_shape=[jax.ShapeDtypeStruct((N, 128), jnp.float32),
                   jax.ShapeDtypeStruct((N, 128), jnp.float32),
                   jax.ShapeDtypeStruct((N, 128), jnp.float32),
                   jax.ShapeDtypeStruct((1, 128), jnp.float32)],
        scratch_shapes=[pltpu.VMEM((1, 128), jnp.float32)],
    )(hprev, num2, den2, bg, gl, bl, wg, acat)


def _tc_post(hprev, num2, den2, bg, gl, bl, wm1, bm1, wm2, bm2):
    return pl.pallas_call(
        _tc_post_body,
        grid=(GRID,),
        in_specs=[_row_spec(), _pair_spec(), _pair_spec()]
                 + [_w_spec(1)] * 3
                 + [pl.BlockSpec((128, 64), lambda i: (0, 0)),
                    pl.BlockSpec((1, 64), lambda i: (0, 0)),
                    pl.BlockSpec((64, 128), lambda i: (0, 0)),
                    _w_spec(1)],
        out_specs=_row_spec(),
        out_shape=jax.ShapeDtypeStruct((N, 128), jnp.float32),
    )(hprev, num2, den2, bg, gl, bl, wm1, bm1, wm2, bm2)


def _sc_edge_body(srcp, dstp, alcat, hp, mrow, znum,
                  num_o, den_o,
                  idxs_v, idxd_v, bs_v, bd_v, rows_v, wbuf_v, mv_v,
                  num_sh, den_sh, semi, semg0, semg1):
    cid = lax.axis_index("c")
    sid = lax.axis_index("s")

    pltpu.sync_copy(znum, num_sh.at[pl.ds(sid * ROWS_PW, ROWS_PW)])

    def zw(j, c2):
        wbuf_v[j] = jnp.zeros((16,), jnp.float32)
        return c2

    lax.fori_loop(0, DP, zw, 0)
    for p in range(ROWS_PW // DP):
        pltpu.sync_copy(wbuf_v.at[pl.ds(0, DP)],
                        den_sh.at[pl.ds(sid * ROWS_PW + p * DP, DP)])
    pltpu.sync_copy(mrow, mv_v)
    plsc.subcore_barrier()

    mvec = mv_v[0, pl.ds(0, 16)]
    base_w = (cid * NSUB + sid) * EPW
    semg = (semg0, semg1)

    def issue_idx(g):
        base = base_w + g * CHUNK
        p = g % 4
        pltpu.async_copy(srcp.at[pl.ds(base, CHUNK)], idxs_v.at[p], semi)
        pltpu.async_copy(dstp.at[pl.ds(base, CHUNK)], idxd_v.at[p], semi)

    def wait_idx():
        pltpu.make_async_copy(srcp.at[pl.ds(0, CHUNK)], idxs_v.at[0],
                              semi).wait()
        pltpu.make_async_copy(dstp.at[pl.ds(0, CHUNK)], idxd_v.at[0],
                              semi).wait()

    def issue_gathers(g, b):
        p = g % 4
        pltpu.async_copy(alcat.at[idxs_v.at[p]], bs_v.at[b], semg[b])
        pltpu.async_copy(alcat.at[idxd_v.at[p]], bd_v.at[b], semg[b])
        pltpu.async_copy(hp.at[idxs_v.at[p]], rows_v.at[b], semg[b])

    def wait_gathers(g, b):
        p = g % 4
        pltpu.make_async_copy(alcat.at[idxs_v.at[p]], bs_v.at[b],
                              semg[b]).wait()
        pltpu.make_async_copy(alcat.at[idxd_v.at[p]], bd_v.at[b],
                              semg[b]).wait()
        pltpu.make_async_copy(hp.at[idxs_v.at[p]], rows_v.at[b],
                              semg[b]).wait()

    def compute_scatter(g, b):
        p = g % 4

        def edge_body(j, c2):
            e = bs_v[b, j, pl.ds(0, 16)] + bd_v[b, j, pl.ds(8, 16)]
            e = jnp.maximum(e, 0.2 * e) - mvec
            w = jnp.exp(e)
            wbuf_v[j] = w
            for h in range(HEADS):
                rows_v[b, j, pl.ds(h * FH, FH)] = (
                    rows_v[b, j, pl.ds(h * FH, FH)] * w[h])
            return c2

        lax.fori_loop(0, CHUNK, edge_body, 0)
        pltpu.sync_copy(rows_v.at[b], num_sh.at[idxd_v.at[p]], add=True)
        pltpu.sync_copy(wbuf_v, den_sh.at[idxd_v.at[p]], add=True)

    def step(g, k):
        b = k % 2
        wait_idx()
        issue_gathers(g + 1, 1 - b)
        issue_idx(g + 2)
        wait_gathers(g, b)
        compute_scatter(g, b)

    issue_idx(0)
    issue_idx(1)
    wait_idx()
    issue_gathers(0, 0)

    def quad(i, carry):
        g0 = i * 4
        step(g0 + 0, 0)
        step(g0 + 1, 1)
        step(g0 + 2, 2)
        step(g0 + 3, 3)
        return carry

    lax.fori_loop(0, NCHUNK // 4 - 1, quad, 0)

    for k in range(4):
        g = NCHUNK - 4 + k
        b = k % 2
        if g + 1 < NCHUNK:
            wait_idx()
            issue_gathers(g + 1, 1 - b)
        if g + 2 < NCHUNK:
            issue_idx(g + 2)
        wait_gathers(g, b)
        compute_scatter(g, b)

    plsc.subcore_barrier()

    pltpu.sync_copy(num_sh.at[pl.ds(sid * ROWS_PW, ROWS_PW)],
                    num_o.at[cid, pl.ds(sid * ROWS_PW, ROWS_PW)])

    def dexp_body(p2, carry):
        row0 = sid * ROWS_PW + p2 * DP
        pltpu.sync_copy(den_sh.at[pl.ds(row0, DP)], wbuf_v.at[pl.ds(0, DP)])

        def drow(j, c2):
            v = wbuf_v[j]
            for h in range(HEADS):
                rows_v[0, j, pl.ds(h * FH, FH)] = jnp.broadcast_to(
                    v[h], (FH,))
            return c2

        lax.fori_loop(0, DP, drow, 0)
        pltpu.sync_copy(rows_v.at[0, pl.ds(0, DP)],
                        den_o.at[cid, pl.ds(row0, DP)])
        return carry

    lax.fori_loop(0, ROWS_PW // DP, dexp_body, 0)


def _sc_edge(srcp, dstp, alcat, hp, mrow, znum):
    mesh = plsc.VectorSubcoreMesh(core_axis_name="c", subcore_axis_name="s")
    kfn = pl.kernel(
        _sc_edge_body,
        mesh=mesh,
        compiler_params=pltpu.CompilerParams(use_tc_tiling_on_sc=False),
        out_type=[jax.ShapeDtypeStruct((NCORE, NP, 128), jnp.float32),
                  jax.ShapeDtypeStruct((NCORE, NP, 128), jnp.float32)],
        scratch_types=[
            pltpu.VMEM((4, CHUNK), jnp.int32),
            pltpu.VMEM((4, CHUNK), jnp.int32),
            pltpu.VMEM((2, CHUNK, 128), jnp.float32),
            pltpu.VMEM((2, CHUNK, 128), jnp.float32),
            pltpu.VMEM((2, CHUNK, 128), jnp.float32),
            pltpu.VMEM((CHUNK, 16), jnp.float32),
            pltpu.VMEM((1, 128), jnp.float32),
            pltpu.VMEM_SHARED((NP, 128), jnp.float32),
            pltpu.VMEM_SHARED((NP, 16), jnp.float32),
            pltpu.SemaphoreType.DMA,
            pltpu.SemaphoreType.DMA,
            pltpu.SemaphoreType.DMA,
        ],
    )
    return kfn(srcp, dstp, alcat, hp, mrow, znum)


_SEL_S = np.zeros((HID, 128), np.float32)
_SEL_D = np.zeros((HID, 128), np.float32)
for _i in range(HID):
    _SEL_S[_i, _i // FH] = 1.0
    _SEL_D[_i, HEADS + _i // FH] = 1.0
_PIM = np.zeros((IN, IN), np.float32)
_PJM = np.zeros((IN, IN), np.float32)
for _p in range(NPAIR):
    _PIM[_IU[_p], _p] = 1.0
    _PJM[_JU[_p], _p] = 1.0


def _prep_acat(a_s, a_d):
    return (a_s.reshape(-1, 1) * jnp.asarray(_SEL_S)
            + a_d.reshape(-1, 1) * jnp.asarray(_SEL_D))


def kernel(x, edge_index, Wc, bc, gc, bcn, Wg0, as0, ad0, bg0, gl0, bl0,
           Wg1, as1, ad1, bg1, gl1, bl1, Wm1, bm1, Wm2, bm2):
    f32 = jnp.float32
    xp = x.astype(f32)
    src = edge_index[0].astype(jnp.int32)
    dst = edge_index[1].astype(jnp.int32)
    srcp = jnp.concatenate([src, jnp.zeros((EP - E,), jnp.int32)])
    dstp = jnp.concatenate([dst, jnp.full((EP - E,), N, jnp.int32)])

    w1 = Wc[0:IN]
    w2 = Wc[IN:2 * IN]
    w3 = jnp.concatenate([Wc[2 * IN:],
                          jnp.zeros((IN - NPAIR, HID), f32)], axis=0)
    pim = jnp.asarray(_PIM)
    pjm = jnp.asarray(_PJM)

    acat0 = _prep_acat(as0, ad0)
    acat1 = _prep_acat(as1, ad1)
    bc_r = bc.reshape(1, -1)
    gc_r = gc.reshape(1, -1)
    bcn_r = bcn.reshape(1, -1)

    znum = jnp.zeros((ROWS_PW, 128), f32)

    h0, hp0, alcat0_full, m0 = _tc_pre(xp, w1, w2, w3, pim, pjm,
                                       bc_r, gc_r, bcn_r, Wg0, acat0)
    num0, den0 = _sc_edge(srcp, dstp, alcat0_full, hp0, m0, znum)

    h1, hp1, alcat1_full, m1 = _tc_mid(h0, num0, den0,
                                       bg0.reshape(1, -1), gl0.reshape(1, -1),
                                       bl0.reshape(1, -1), Wg1, acat1)
    num1, den1 = _sc_edge(srcp, dstp, alcat1_full, hp1, m1, znum)

    wm2p = jnp.concatenate([Wm2, jnp.zeros((HID // 2, 128 - OUT), f32)], 1)
    bm2p = jnp.concatenate([bm2, jnp.zeros((128 - OUT,), f32)]).reshape(1, -1)
    out128 = _tc_post(h1, num1, den1,
                      bg1.reshape(1, -1), gl1.reshape(1, -1),
                      bl1.reshape(1, -1), Wm1, bm1.reshape(1, -1),
                      wm2p, bm2p)
    return out128[:N, :OUT]


# speedup vs baseline: 1.2524x; 1.2524x over previous
import functools

import jax
import jax.numpy as jnp
import numpy as np
from jax import lax
from jax.experimental import pallas as pl
from jax.experimental.pallas import tpu as pltpu
from jax.experimental.pallas import tpu_sc as plsc

N = 10000
E = 160000
IN = 128
HID = 128
HEADS = 8
FH = HID // HEADS
K = 8
OUT = 2

NP = 10240
EP = 163840
BLK = 200
GRID = N // BLK
NCORE = 2
NSUB = 16
CHUNK = 40
EPW = EP // (NCORE * NSUB)
NCHUNK = EPW // CHUNK
ROWS_PW = NP // NSUB

_IU, _JU = np.triu_indices(K, 1)
NPAIR = len(_IU)


def _ln(h, g, b):
    m = jnp.mean(h, axis=-1, keepdims=True)
    v = jnp.mean((h - m) * (h - m), axis=-1, keepdims=True)
    return (h - m) / jnp.sqrt(v + 1e-5) * g + b


def _elu(h):
    return jnp.where(h > 0, h, jnp.exp(h) - 1.0)


def _attn_tail(hp, acat, step, macc_ref, m_ref):
    alcat = jnp.dot(hp, acat, preferred_element_type=jnp.float32)
    bmax = jnp.max(alcat, axis=0, keepdims=True)

    @pl.when(step == 0)
    def _():
        macc_ref[...] = bmax

    @pl.when(step > 0)
    def _():
        macc_ref[...] = jnp.maximum(macc_ref[...], bmax)

    mc = macc_ref[...]
    ms = jnp.max(mc[:, 0:8]) + jnp.max(mc[:, 8:16])
    ms = jnp.maximum(ms, 0.0)
    m_ref[...] = jnp.full((1, 128), ms, jnp.float32)
    return alcat


def _tc_pre_body(x_ref, w1_ref, w2_ref, w3_ref, pi_ref, pj_ref, bc_ref,
                 gc_ref, bcn_ref, wg_ref, acat_ref,
                 h_ref, hp_ref, alcat_ref, m_ref, macc_ref):
    i = pl.program_id(0)
    x = x_ref[...]
    cosx = jnp.cos(jnp.pi * x)
    sinx = jnp.sin(jnp.pi * x)
    pre = jnp.dot(cosx, w1_ref[...], preferred_element_type=jnp.float32)
    pre += jnp.dot(sinx, w2_ref[...], preferred_element_type=jnp.float32)
    si = jnp.dot(sinx, pi_ref[...], preferred_element_type=jnp.float32)
    sj = jnp.dot(sinx, pj_ref[...], preferred_element_type=jnp.float32)
    pre += jnp.dot(si * sj, w3_ref[...], preferred_element_type=jnp.float32)
    pre += bc_ref[...]
    h = _elu(_ln(pre, gc_ref[...], bcn_ref[...]))
    h_ref[...] = h
    hp = jnp.dot(h, wg_ref[...], preferred_element_type=jnp.float32)
    hp_ref[...] = hp
    alcat_ref[...] = _attn_tail(hp, acat_ref[...], i, macc_ref, m_ref)


def _tc_mid_body(hprev_ref, num_ref, den_ref, bg_ref, gl_ref, bl_ref,
                 wg_ref, acat_ref,
                 h_ref, hp_ref, alcat_ref, m_ref, macc_ref):
    i = pl.program_id(0)
    nsum = num_ref[0] + num_ref[1]
    dsum = den_ref[0] + den_ref[1]
    gat = nsum / (dsum + 1e-16) + bg_ref[...]
    h = _elu(_ln(gat, gl_ref[...], bl_ref[...])) + hprev_ref[...]
    h_ref[...] = h
    hp = jnp.dot(h, wg_ref[...], preferred_element_type=jnp.float32)
    hp_ref[...] = hp
    alcat_ref[...] = _attn_tail(hp, acat_ref[...], i, macc_ref, m_ref)


def _tc_post_body(hprev_ref, num_ref, den_ref, bg_ref, gl_ref, bl_ref,
                  wm1_ref, bm1_ref, wm2_ref, bm2_ref, out_ref):
    nsum = num_ref[0] + num_ref[1]
    dsum = den_ref[0] + den_ref[1]
    gat = nsum / (dsum + 1e-16) + bg_ref[...]
    h = _elu(_ln(gat, gl_ref[...], bl_ref[...])) + hprev_ref[...]
    t = _elu(jnp.dot(h, wm1_ref[...], preferred_element_type=jnp.float32)
             + bm1_ref[...])
    out_ref[...] = (jnp.dot(t, wm2_ref[...], preferred_element_type=jnp.float32)
                    + bm2_ref[...])


def _row_spec():
    return pl.BlockSpec((BLK, 128), lambda i: (i, 0))


def _w_spec(r, c=128):
    return pl.BlockSpec((r, c), lambda i: (0, 0))


def _pair_spec():
    return pl.BlockSpec((2, BLK, 128), lambda i: (0, i, 0))


def _tc_pre(x, w1, w2, w3, pim, pjm, bc, gc, bcn, wg, acat):
    return pl.pallas_call(
        _tc_pre_body,
        grid=(GRID,),
        in_specs=[_row_spec()] + [_w_spec(128)] * 5 + [_w_spec(1)] * 3
                 + [_w_spec(128)] * 2,
        out_specs=[_row_spec(), _row_spec(), _row_spec(),
                   pl.BlockSpec((1, 128), lambda i: (0, 0))],
        out_shape=[jax.ShapeDtypeStruct((N, 128), jnp.float32),
                   jax.ShapeDtypeStruct((N, 128), jnp.float32),
                   jax.ShapeDtypeStruct((N, 128), jnp.float32),
                   jax.ShapeDtypeStruct((1, 128), jnp.float32)],
        scratch_shapes=[pltpu.VMEM((1, 128), jnp.float32)],
    )(x, w1, w2, w3, pim, pjm, bc, gc, bcn, wg, acat)


def _tc_mid(hprev, num2, den2, bg, gl, bl, wg, acat):
    return pl.pallas_call(
        _tc_mid_body,
        grid=(GRID,),
        in_specs=[_row_spec(), _pair_spec(), _pair_spec()]
                 + [_w_spec(1)] * 3 + [_w_spec(128)] * 2,
        out_specs=[_row_spec(), _row_spec(), _row_spec(),
                   pl.BlockSpec((1, 128), lambda i: (0, 0))],
        out_shape=[jax.ShapeDtypeStruct((N, 128), jnp.float32),
                   jax.ShapeDtypeStruct((N, 128), jnp.float32),
                   jax.ShapeDtypeStruct((N, 128), jnp.float32),
                   jax.ShapeDtypeStruct((1, 128), jnp.float32)],
        scratch_shapes=[pltpu.VMEM((1, 128), jnp.float32)],
    )(hprev, num2, den2, bg, gl, bl, wg, acat)


def _tc_post(hprev, num2, den2, bg, gl, bl, wm1, bm1, wm2, bm2):
    return pl.pallas_call(
        _tc_post_body,
        grid=(GRID,),
        in_specs=[_row_spec(), _pair_spec(), _pair_spec()]
                 + [_w_spec(1)] * 3
                 + [pl.BlockSpec((128, 64), lambda i: (0, 0)),
                    pl.BlockSpec((1, 64), lambda i: (0, 0)),
                    pl.BlockSpec((64, 128), lambda i: (0, 0)),
                    _w_spec(1)],
        out_specs=_row_spec(),
        out_shape=jax.ShapeDtypeStruct((N, 128), jnp.float32),
    )(hprev, num2, den2, bg, gl, bl, wm1, bm1, wm2, bm2)


def _sc_edge_body(srcp, dstp, alcat, hp, mrow, znum,
                  num_o, den_o,
                  idxs_v, idxd_v, bs_v, bd_v, rows_v, wbuf_v, mv_v,
                  num_sh, den_sh, semi, semg0, semg1):
    cid = lax.axis_index("c")
    sid = lax.axis_index("s")

    pltpu.sync_copy(znum, num_sh.at[pl.ds(sid * ROWS_PW, ROWS_PW)])

    def zw(j, c2):
        wbuf_v[0, j] = jnp.zeros((16,), jnp.float32)
        return c2

    lax.fori_loop(0, CHUNK, zw, 0)
    for p in range(ROWS_PW // CHUNK):
        pltpu.sync_copy(wbuf_v.at[0],
                        den_sh.at[pl.ds(sid * ROWS_PW + p * CHUNK, CHUNK)])
    pltpu.sync_copy(mrow, mv_v)
    plsc.subcore_barrier()

    mvec = mv_v[0, pl.ds(0, 16)]
    base_w = (cid * NSUB + sid) * EPW
    semg = (semg0, semg1)

    def issue_idx(g):
        base = base_w + g * CHUNK
        p = g % 4
        pltpu.async_copy(srcp.at[pl.ds(base, CHUNK)], idxs_v.at[p], semi)
        pltpu.async_copy(dstp.at[pl.ds(base, CHUNK)], idxd_v.at[p], semi)

    def wait_idx():
        pltpu.make_async_copy(srcp.at[pl.ds(0, CHUNK)], idxs_v.at[0],
                              semi).wait()
        pltpu.make_async_copy(dstp.at[pl.ds(0, CHUNK)], idxd_v.at[0],
                              semi).wait()

    def issue_gathers(g, b):
        p = g % 4
        pltpu.async_copy(alcat.at[idxs_v.at[p]], bs_v.at[b], semg[b])
        pltpu.async_copy(alcat.at[idxd_v.at[p]], bd_v.at[b], semg[b])
        pltpu.async_copy(hp.at[idxs_v.at[p]], rows_v.at[b], semg[b])

    def wait_gathers(g, b):
        p = g % 4
        pltpu.make_async_copy(alcat.at[idxs_v.at[p]], bs_v.at[b],
                              semg[b]).wait()
        pltpu.make_async_copy(alcat.at[idxd_v.at[p]], bd_v.at[b],
                              semg[b]).wait()
        pltpu.make_async_copy(hp.at[idxs_v.at[p]], rows_v.at[b],
                              semg[b]).wait()

    def compute_scatter(g, b):
        p = g % 4

        def edge_body(j, c2):
            e = bs_v[b, j, pl.ds(0, 16)] + bd_v[b, j, pl.ds(8, 16)]
            e = jnp.maximum(e, 0.2 * e) - mvec
            w = jnp.exp(e)
            wbuf_v[b, j] = w
            for h in range(HEADS):
                rows_v[b, j, pl.ds(h * FH, FH)] = (
                    rows_v[b, j, pl.ds(h * FH, FH)] * w[h])
            return c2

        lax.fori_loop(0, CHUNK, edge_body, 0)
        pltpu.sync_copy(rows_v.at[b], num_sh.at[idxd_v.at[p]], add=True)
        pltpu.sync_copy(wbuf_v.at[b], den_sh.at[idxd_v.at[p]], add=True)

    def step(g, k):
        b = k % 2
        wait_idx()
        issue_gathers(g + 1, 1 - b)
        issue_idx(g + 2)
        wait_gathers(g, b)
        compute_scatter(g, b)

    issue_idx(0)
    issue_idx(1)
    wait_idx()
    issue_gathers(0, 0)

    def quad(i, carry):
        g0 = i * 4
        step(g0 + 0, 0)
        step(g0 + 1, 1)
        step(g0 + 2, 2)
        step(g0 + 3, 3)
        return carry

    lax.fori_loop(0, NCHUNK // 4 - 1, quad, 0)

    for k in range(4):
        g = NCHUNK - 4 + k
        b = k % 2
        if g + 1 < NCHUNK:
            wait_idx()
            issue_gathers(g + 1, 1 - b)
        if g + 2 < NCHUNK:
            issue_idx(g + 2)
        wait_gathers(g, b)
        compute_scatter(g, b)

    plsc.subcore_barrier()

    pltpu.sync_copy(num_sh.at[pl.ds(sid * ROWS_PW, ROWS_PW)],
                    num_o.at[cid, pl.ds(sid * ROWS_PW, ROWS_PW)])

    def dexp_body(p2, carry):
        row0 = sid * ROWS_PW + p2 * CHUNK
        pltpu.sync_copy(den_sh.at[pl.ds(row0, CHUNK)], wbuf_v.at[0])

        def drow(j, c2):
            v = wbuf_v[0, j]
            for h in range(HEADS):
                rows_v[0, j, pl.ds(h * FH, FH)] = jnp.broadcast_to(
                    v[h], (FH,))
            return c2

        lax.fori_loop(0, CHUNK, drow, 0)
        pltpu.sync_copy(rows_v.at[0], den_o.at[cid, pl.ds(row0, CHUNK)])
        return carry

    lax.fori_loop(0, ROWS_PW // CHUNK, dexp_body, 0)


def _sc_edge(srcp, dstp, alcat, hp, mrow, znum):
    mesh = plsc.VectorSubcoreMesh(core_axis_name="c", subcore_axis_name="s")
    kfn = pl.kernel(
        _sc_edge_body,
        mesh=mesh,
        compiler_params=pltpu.CompilerParams(use_tc_tiling_on_sc=False),
        out_type=[jax.ShapeDtypeStruct((NCORE, NP, 128), jnp.float32),
                  jax.ShapeDtypeStruct((NCORE, NP, 128), jnp.float32)],
        scratch_types=[
            pltpu.VMEM((4, CHUNK), jnp.int32),
            pltpu.VMEM((4, CHUNK), jnp.int32),
            pltpu.VMEM((2, CHUNK, 128), jnp.float32),
            pltpu.VMEM((2, CHUNK, 128), jnp.float32),
            pltpu.VMEM((2, CHUNK, 128), jnp.float32),
            pltpu.VMEM((2, CHUNK, 16), jnp.float32),
            pltpu.VMEM((1, 128), jnp.float32),
            pltpu.VMEM_SHARED((NP, 128), jnp.float32),
            pltpu.VMEM_SHARED((NP, 16), jnp.float32),
            pltpu.SemaphoreType.DMA,
            pltpu.SemaphoreType.DMA,
            pltpu.SemaphoreType.DMA,
        ],
    )
    return kfn(srcp, dstp, alcat, hp, mrow, znum)


_SEL_S = np.zeros((HID, 128), np.float32)
_SEL_D = np.zeros((HID, 128), np.float32)
for _i in range(HID):
    _SEL_S[_i, _i // FH] = 1.0
    _SEL_D[_i, HEADS + _i // FH] = 1.0
_PIM = np.zeros((IN, IN), np.float32)
_PJM = np.zeros((IN, IN), np.float32)
for _p in range(NPAIR):
    _PIM[_IU[_p], _p] = 1.0
    _PJM[_JU[_p], _p] = 1.0


def _prep_acat(a_s, a_d):
    return (a_s.reshape(-1, 1) * jnp.asarray(_SEL_S)
            + a_d.reshape(-1, 1) * jnp.asarray(_SEL_D))


def kernel(x, edge_index, Wc, bc, gc, bcn, Wg0, as0, ad0, bg0, gl0, bl0,
           Wg1, as1, ad1, bg1, gl1, bl1, Wm1, bm1, Wm2, bm2):
    f32 = jnp.float32
    xp = x.astype(f32)
    src = edge_index[0].astype(jnp.int32)
    dst = edge_index[1].astype(jnp.int32)
    srcp = jnp.concatenate([src, jnp.zeros((EP - E,), jnp.int32)])
    dstp = jnp.concatenate([dst, jnp.full((EP - E,), N, jnp.int32)])

    w1 = Wc[0:IN]
    w2 = Wc[IN:2 * IN]
    w3 = jnp.concatenate([Wc[2 * IN:],
                          jnp.zeros((IN - NPAIR, HID), f32)], axis=0)
    pim = jnp.asarray(_PIM)
    pjm = jnp.asarray(_PJM)

    acat0 = _prep_acat(as0, ad0)
    acat1 = _prep_acat(as1, ad1)
    bc_r = bc.reshape(1, -1)
    gc_r = gc.reshape(1, -1)
    bcn_r = bcn.reshape(1, -1)

    znum = jnp.zeros((ROWS_PW, 128), f32)

    h0, hp0, alcat0_full, m0 = _tc_pre(xp, w1, w2, w3, pim, pjm,
                                       bc_r, gc_r, bcn_r, Wg0, acat0)
    num0, den0 = _sc_edge(srcp, dstp, alcat0_full, hp0, m0, znum)

    h1, hp1, alcat1_full, m1 = _tc_mid(h0, num0, den0,
                                       bg0.reshape(1, -1), gl0.reshape(1, -1),
                                       bl0.reshape(1, -1), Wg1, acat1)
    num1, den1 = _sc_edge(srcp, dstp, alcat1_full, hp1, m1, znum)

    wm2p = jnp.concatenate([Wm2, jnp.zeros((HID // 2, 128 - OUT), f32)], 1)
    bm2p = jnp.concatenate([bm2, jnp.zeros((128 - OUT,), f32)]).reshape(1, -1)
    out128 = _tc_post(h1, num1, den1,
                      bg1.reshape(1, -1), gl1.reshape(1, -1),
                      bl1.reshape(1, -1), Wm1, bm1.reshape(1, -1),
                      wm2p, bm2p)
    return out128[:N, :OUT]
